# Initial kernel scaffold; baseline (speedup 1.0000x reference)
import jax
import jax.numpy as jnp
from jax.experimental import pallas as pl


def kernel(xyz, feat, idx, W, b):
    raise NotImplementedError("write your pallas kernel here")


# speedup vs baseline: 20.0114x; 20.0114x over previous
import functools

import jax
import jax.numpy as jnp
from jax import lax
from jax.experimental import pallas as pl
from jax.experimental.pallas import tpu as pltpu
from jax.experimental.pallas import tpu_sc as plsc

B, N, K, D = 4, 4096, 16, 128
BN = B * N
BNK = B * N * K
NW = 32
ROWS_PW = BNK // NW
CH = 512
PB = 64


def _sc_gather_body(tx_hbm, ty_hbm, tz_hbm, feat_hbm, gidx_hbm,
                    px_out, py_out, pz_out, f_out,
                    txv, tyv, tzv, idxv, pxb, pyb, pzb, fbuf,
                    sem_i, sem_f):
    wid = lax.axis_index("s") * 2 + lax.axis_index("c")
    base0 = wid * ROWS_PW
    pltpu.sync_copy(tx_hbm, txv)
    pltpu.sync_copy(ty_hbm, tyv)
    pltpu.sync_copy(tz_hbm, tzv)

    def body(j, carry):
        base = base0 + j * CH
        pltpu.async_copy(gidx_hbm.at[pl.ds(base, CH)], idxv, sem_i).wait()
        cp_f = pltpu.async_copy(feat_hbm.at[idxv], fbuf, sem_f)

        def inner(i, c):
            s = pl.ds(i * 16, 16)
            v = idxv[s]
            pxb[s] = plsc.load_gather(txv, [v])
            pyb[s] = plsc.load_gather(tyv, [v])
            pzb[s] = plsc.load_gather(tzv, [v])
            return c

        lax.fori_loop(0, CH // 16, inner, 0)
        cp_f.wait()
        pltpu.sync_copy(pxb, px_out.at[pl.ds(base, CH)])
        pltpu.sync_copy(pyb, py_out.at[pl.ds(base, CH)])
        pltpu.sync_copy(pzb, pz_out.at[pl.ds(base, CH)])
        pltpu.sync_copy(fbuf, f_out.at[pl.ds(base, CH)])
        return carry

    lax.fori_loop(0, ROWS_PW // CH, body, 0)


def _sc_gather(tx, ty, tz, feat2d, gidx):
    mesh = plsc.VectorSubcoreMesh(core_axis_name="c", subcore_axis_name="s")
    fn = functools.partial(
        pl.kernel,
        mesh=mesh,
        compiler_params=pltpu.CompilerParams(needs_layout_passes=False),
        out_type=[
            jax.ShapeDtypeStruct((BNK,), jnp.float32),
            jax.ShapeDtypeStruct((BNK,), jnp.float32),
            jax.ShapeDtypeStruct((BNK,), jnp.float32),
            jax.ShapeDtypeStruct((BNK, D), jnp.float32),
        ],
        scratch_types=[
            pltpu.VMEM((BN,), jnp.float32),
            pltpu.VMEM((BN,), jnp.float32),
            pltpu.VMEM((BN,), jnp.float32),
            pltpu.VMEM((CH,), jnp.int32),
            pltpu.VMEM((CH,), jnp.float32),
            pltpu.VMEM((CH,), jnp.float32),
            pltpu.VMEM((CH,), jnp.float32),
            pltpu.VMEM((CH, D), jnp.float32),
            pltpu.SemaphoreType.DMA,
            pltpu.SemaphoreType.DMA,
        ],
    )(_sc_gather_body)
    return fn(tx, ty, tz, feat2d, gidx)


def _tc_body(xyz_ref, px_ref, py_ref, pz_ref, f_ref, w_ref, b_ref, o_ref):
    w = w_ref[...]
    wa = w[0:3] - w[6:9]
    wc0 = w[3] + w[6]
    wc1 = w[4] + w[7]
    wc2 = w[5] + w[8]
    w9 = w[9]
    bb = b_ref[...][0]
    wa16 = jnp.concatenate([wa, jnp.zeros((13, D), jnp.float32)], axis=0)

    cen = xyz_ref[...]
    px = px_ref[...]
    py = py_ref[...]
    pz = pz_ref[...]
    dx = px - cen[:, 0:1]
    dy = py - cen[:, 1:2]
    dz = pz - cen[:, 2:3]
    norm = jnp.sqrt(dx * dx + dy * dy + dz * dz)

    dn = (((1,), (0,)), ((), ()))
    cen_a = lax.dot_general(cen, wa16, dn,
                            precision=lax.Precision.HIGHEST)
    r = (cen_a[:, None, :] + bb
         + px[:, :, None] * wc0 + py[:, :, None] * wc1
         + pz[:, :, None] * wc2 + norm[:, :, None] * w9)
    o_ref[:, :, 0:D] = jnp.maximum(r, 0.0)
    o_ref[:, :, D:2 * D] = f_ref[...]


def _tc_assemble(xyz16, px_g, py_g, pz_g, f_g, W, b2d):
    grid = (BN // PB,)
    pk_spec = pl.BlockSpec((PB, K), lambda i: (i, 0))
    return pl.pallas_call(
        _tc_body,
        grid=grid,
        in_specs=[
            pl.BlockSpec((PB, 16), lambda i: (i, 0)),
            pk_spec, pk_spec, pk_spec,
            pl.BlockSpec((PB, K, D), lambda i: (i, 0, 0)),
            pl.BlockSpec((10, D), lambda i: (0, 0)),
            pl.BlockSpec((1, D), lambda i: (0, 0)),
        ],
        out_specs=pl.BlockSpec((PB, K, 2 * D), lambda i: (i, 0, 0)),
        out_shape=jax.ShapeDtypeStruct((BN, K, 2 * D), jnp.float32),
    )(xyz16, px_g, py_g, pz_g, f_g, W, b2d)


def kernel(xyz, feat, idx, W, b):
    xyz2 = xyz.reshape(BN, 3)
    xyz16 = jnp.pad(xyz2, ((0, 0), (0, 13)))
    tx = xyz2[:, 0]
    ty = xyz2[:, 1]
    tz = xyz2[:, 2]
    feat2d = feat.reshape(BN, D)
    gidx = (idx + (jnp.arange(B, dtype=idx.dtype) * N)[:, None, None])
    gidx = gidx.reshape(BNK)
    px_g, py_g, pz_g, f_g = _sc_gather(tx, ty, tz, feat2d, gidx)
    out = _tc_assemble(xyz16, px_g.reshape(BN, K), py_g.reshape(BN, K),
                       pz_g.reshape(BN, K), f_g.reshape(BN, K, D),
                       W, b.reshape(1, D))
    return out.reshape(B, N, K, 2 * D)
